# baseline (speedup 1.0000x reference)
import functools

import jax
import jax.numpy as jnp
from jax import lax
from jax.experimental import pallas as pl
from jax.experimental.pallas import tpu as pltpu
from jax.experimental.pallas import tpu_sc as plsc

NC = 2
NS = 16
NW = NC * NS

B = 4
T = 2048
D = 768
LANES = 16
DV = D // LANES

TW = T // NW
CW = 8
NCHUNK = TW // CW
NBUF = 3


def _emb_body(
    idx_hbm, pos_hbm, tok_hbm, out_hbm,
    idx_v, pos_v, rows_v,
    isem, gsem0, gsem1, wsem0, wsem1, psem0, psem1,
):
    wid = lax.axis_index("s") * NC + lax.axis_index("c")
    t0 = wid * TW
    gsems = (gsem0, gsem1)
    wsems = (wsem0, wsem1)
    psems = (psem0, psem1)

    def idx_cp(h, b):
        return pltpu.async_copy(
            idx_hbm.at[pl.ds(b * T + t0 + h * CW, CW)],
            idx_v.at[pl.ds(h * (B * CW) + b * CW, CW)],
            isem,
        )

    i0 = [idx_cp(0, b) for b in range(B)]
    for cp in i0:
        cp.wait()

    def gathers(h):
        return [
            pltpu.async_copy(
                tok_hbm.at[idx_v.at[pl.ds(h * (B * CW), B * CW)]],
                rows_v.at[h % NBUF],
                gsems[h % 2],
            )
        ]

    def pos_copy(h):
        return pltpu.async_copy(
            pos_hbm.at[pl.ds(t0 + h * CW, CW)], pos_v.at[h % 2], psems[h % 2]
        )

    g = {0: gathers(0)}
    p = {0: pos_copy(0)}
    irest = [idx_cp(h, b) for h in range(1, NCHUNK) for b in range(B)]
    for cp in irest:
        cp.wait()
    w = {}
    for h in range(NCHUNK):
        if h + 1 < NCHUNK:
            prev = h + 1 - NBUF
            if prev >= 0:
                for cp in w[prev]:
                    cp.wait()
            g[h + 1] = gathers(h + 1)
            p[h + 1] = pos_copy(h + 1)
        for cp in g[h]:
            cp.wait()
        p[h].wait()

        buf = rows_v.at[h % NBUF]
        pb = h % 2

        def j_body(j, _):
            for u in range(2):
                sl = pl.ds((j * 2 + u) * LANES, LANES)
                for r in range(CW):
                    pv = pos_v[pb, r, sl]
                    for b in range(B):
                        plsc.addupdate(buf.at[b * CW + r, sl], pv)
            return _

        lax.fori_loop(0, DV // 2, j_body, 0)

        w[h] = [
            pltpu.async_copy(
                buf.at[pl.ds(b * CW, CW)],
                out_hbm.at[pl.ds(b * T + t0 + h * CW, CW)],
                wsems[h % 2],
            )
            for b in range(B)
        ]
    for h in range(max(0, NCHUNK - NBUF + 1), NCHUNK):
        for cp in w[h]:
            cp.wait()


@functools.lru_cache(maxsize=None)
def _emb_call():
    return functools.partial(
        pl.kernel,
        out_type=jax.ShapeDtypeStruct((B * T, D), jnp.float32),
        mesh=plsc.VectorSubcoreMesh(
            core_axis_name="c", subcore_axis_name="s", num_cores=NC, num_subcores=NS
        ),
        scratch_types=[
            pltpu.VMEM((B * TW,), jnp.int32),
            pltpu.VMEM((2, CW, D), jnp.float32),
            pltpu.VMEM((NBUF, B * CW, D), jnp.float32),
            pltpu.SemaphoreType.DMA,
            pltpu.SemaphoreType.DMA,
            pltpu.SemaphoreType.DMA,
            pltpu.SemaphoreType.DMA,
            pltpu.SemaphoreType.DMA,
            pltpu.SemaphoreType.DMA,
            pltpu.SemaphoreType.DMA,
        ],
    )(_emb_body)


@jax.jit
def kernel(idx, tok_emb, pos_emb):
    b, t = idx.shape
    idx_flat = idx.astype(jnp.int32).reshape(b * t)
    pos2d = pos_emb.reshape(pos_emb.shape[1], pos_emb.shape[2])[:t]
    out = _emb_call()(idx_flat, pos2d, tok_emb)
    return out.reshape(b, t, pos_emb.shape[2])


# speedup vs baseline: 1.0931x; 1.0931x over previous
import functools

import jax
import jax.numpy as jnp
from jax import lax
from jax.experimental import pallas as pl
from jax.experimental.pallas import tpu as pltpu
from jax.experimental.pallas import tpu_sc as plsc

NC = 2
NS = 16
NW = NC * NS

B = 4
T = 2048
D = 768
LANES = 16
DV = D // LANES

TW = T // NW
CW = 8
NCHUNK = TW // CW
NBUF = 3


def _emb_body(
    idx_hbm, pos_hbm, tok_hbm, out_hbm,
    idx_v, pos_v, rows_v,
    isem, gsem0, gsem1, wsem0, wsem1, psem0, psem1,
):
    wid = lax.axis_index("s") * NC + lax.axis_index("c")
    t0 = wid * TW
    gsems = (gsem0, gsem1)
    wsems = (wsem0, wsem1)
    psems = (psem0, psem1)

    def idx_cp(h, b):
        return pltpu.async_copy(
            idx_hbm.at[pl.ds(b * T + t0 + h * CW, CW)],
            idx_v.at[pl.ds(h * (B * CW) + b * CW, CW)],
            isem,
        )

    i0 = [idx_cp(0, b) for b in range(B)]
    for cp in i0:
        cp.wait()

    def gathers(h):
        return [
            pltpu.async_copy(
                tok_hbm.at[idx_v.at[pl.ds(h * (B * CW), B * CW)]],
                rows_v.at[h % NBUF],
                gsems[h % 2],
            )
        ]

    def pos_copy(h):
        return pltpu.async_copy(
            pos_hbm.at[pl.ds(t0 + h * CW, CW)], pos_v.at[h % 2], psems[h % 2]
        )

    g = {0: gathers(0)}
    p = {0: pos_copy(0)}
    irest = [idx_cp(h, b) for h in range(1, NCHUNK) for b in range(B)]
    for cp in irest:
        cp.wait()
    w = {}
    for h in range(NCHUNK):
        if h + 1 < NCHUNK:
            prev = h + 1 - NBUF
            if prev >= 0:
                for cp in w[prev]:
                    cp.wait()
            g[h + 1] = gathers(h + 1)
            p[h + 1] = pos_copy(h + 1)
        for cp in g[h]:
            cp.wait()
        p[h].wait()

        buf = rows_v.at[h % NBUF]
        pb = h % 2

        def j_body(j, _):
            sl = pl.ds(j * LANES, LANES)
            for r in range(CW):
                pv = pos_v[pb, r, sl]
                for b in range(B):
                    plsc.addupdate(buf.at[b * CW + r, sl], pv)
            return _

        lax.fori_loop(0, DV, j_body, 0)

        w[h] = [
            pltpu.async_copy(
                buf.at[pl.ds(b * CW, CW)],
                out_hbm.at[pl.ds(b * T + t0 + h * CW, CW)],
                wsems[h % 2],
            )
            for b in range(B)
        ]
    for h in range(max(0, NCHUNK - NBUF + 1), NCHUNK):
        for cp in w[h]:
            cp.wait()


@functools.lru_cache(maxsize=None)
def _emb_call():
    return functools.partial(
        pl.kernel,
        out_type=jax.ShapeDtypeStruct((B * T, D), jnp.float32),
        mesh=plsc.VectorSubcoreMesh(
            core_axis_name="c", subcore_axis_name="s", num_cores=NC, num_subcores=NS
        ),
        scratch_types=[
            pltpu.VMEM((B * TW,), jnp.int32),
            pltpu.VMEM((2, CW, D), jnp.float32),
            pltpu.VMEM((NBUF, B * CW, D), jnp.float32),
            pltpu.SemaphoreType.DMA,
            pltpu.SemaphoreType.DMA,
            pltpu.SemaphoreType.DMA,
            pltpu.SemaphoreType.DMA,
            pltpu.SemaphoreType.DMA,
            pltpu.SemaphoreType.DMA,
            pltpu.SemaphoreType.DMA,
        ],
    )(_emb_body)


@jax.jit
def kernel(idx, tok_emb, pos_emb):
    b, t = idx.shape
    idx_flat = idx.astype(jnp.int32).reshape(b * t)
    pos2d = pos_emb.reshape(pos_emb.shape[1], pos_emb.shape[2])[:t]
    out = _emb_call()(idx_flat, pos2d, tok_emb)
    return out.reshape(b, t, pos_emb.shape[2])
